# baseline (speedup 1.0000x reference)
import functools

import jax
import jax.numpy as jnp
from jax import lax
from jax.experimental import pallas as pl
from jax.experimental.pallas import tpu as pltpu
from jax.experimental.pallas import tpu_sc as plsc

_LANES = 16


@functools.lru_cache(maxsize=None)
def _make_kernel(B, L, V, D):
    info = plsc.get_sparse_core_info()
    NC, NS = info.num_cores, info.num_subcores
    NW = NC * NS
    assert L % NW == 0 and D % _LANES == 0
    LW = L // NW
    C = min(32, LW)
    assert LW % C == 0 and C % 8 == 0
    n_chunks = LW // C
    n_vec = D // _LANES
    NSLOT = 3
    steps = [(ci, b) for ci in range(n_chunks) for b in range(B)]
    T = len(steps)

    mesh = plsc.VectorSubcoreMesh(core_axis_name="c", subcore_axis_name="s")

    @functools.partial(
        pl.kernel,
        mesh=mesh,
        out_type=jax.ShapeDtypeStruct((B, L, D), jnp.float32),
        scratch_types=[
            pltpu.VMEM((B, LW), jnp.int32),
            pltpu.VMEM((C, D), jnp.float32),
            pltpu.VMEM((NSLOT, C, D), jnp.float32),
            pltpu.SemaphoreType.DMA,
            pltpu.SemaphoreType.DMA,
            pltpu.SemaphoreType.DMA,
            pltpu.SemaphoreType.DMA,
        ],
    )
    def emb(x_hbm, table_hbm, enc_hbm, out_hbm,
            idx_v, enc_v, rows_v, gsem, esem, isem, ssem):
        wid = lax.axis_index("s") * NC + lax.axis_index("c")
        l0 = wid * LW

        for b in range(B):
            pltpu.async_copy(x_hbm.at[b, pl.ds(l0, LW)], idx_v.at[b], isem)
        for b in range(B):
            pltpu.make_async_copy(
                x_hbm.at[b, pl.ds(l0, LW)], idx_v.at[b], isem
            ).wait()

        def fire(t):
            ci, b = steps[t]
            pltpu.async_copy(
                table_hbm.at[idx_v.at[b, pl.ds(ci * C, C)]],
                rows_v.at[t % NSLOT],
                gsem,
            )

        def store_descr(t):
            ci, b = steps[t]
            return pltpu.make_async_copy(
                rows_v.at[t % NSLOT],
                out_hbm.at[b, pl.ds(l0 + ci * C, C)],
                ssem,
            )

        pltpu.async_copy(enc_hbm.at[pl.ds(l0, C)], enc_v, esem)
        fire(0)

        for t, (ci, b) in enumerate(steps):
            slot = t % NSLOT
            if t + 1 < T:
                if t >= 2:
                    store_descr(t - 2).wait()
                fire(t + 1)
            if b == 0 and ci > 0:
                pltpu.async_copy(enc_hbm.at[pl.ds(l0 + ci * C, C)], enc_v, esem)
            if b == 0:
                pltpu.make_async_copy(
                    enc_hbm.at[pl.ds(l0, C)], enc_v, esem
                ).wait()
            pltpu.make_async_copy(
                table_hbm.at[idx_v.at[b, pl.ds(ci * C, C)]],
                rows_v.at[slot],
                gsem,
            ).wait()

            def row_body(r, _, slot=slot):
                for j in range(n_vec):
                    sl = pl.ds(j * _LANES, _LANES)
                    plsc.addupdate(rows_v.at[slot, r, sl], enc_v[r, sl])
                return 0

            lax.fori_loop(0, C, row_body, 0)
            store_descr(t).start()

        for t in range(max(0, T - 3), T):
            store_descr(t).wait()

    return emb


def kernel(x, table, enc):
    B, L = x.shape
    V, D = table.shape
    emb = _make_kernel(B, L, V, D)
    return emb(x.astype(jnp.int32), table, enc)
